# baseline (speedup 1.0000x reference)
import functools

import jax
import jax.numpy as jnp
import numpy as np
from jax import lax
from jax.experimental import pallas as pl
from jax.experimental.pallas import tpu as pltpu
from jax.experimental.pallas import tpu_sc as plsc

B = 16384
NUM_DENSE = 13
N_CAT = 26
VOCAB = 100000
EMB_DIM = 64
NF = N_CAT + 1
NFP = 32
GW = 128
BR = 512


def _sc_gather(table, idx):
    n_idx = idx.shape[1]
    mesh = plsc.VectorSubcoreMesh(core_axis_name="c", subcore_axis_name="s")

    @functools.partial(
        pl.kernel,
        mesh=mesh,
        out_type=jax.ShapeDtypeStruct((n_idx, 2 * EMB_DIM), jnp.float32),
    )
    def k(table_hbm, idx_hbm, out_hbm):
        def body(i_vmem, o_vmem):
            pltpu.sync_copy(table_hbm.at[i_vmem.at[0]], o_vmem)

        pltpu.emit_pipeline(
            body,
            grid=(n_idx // GW,),
            in_specs=[pl.BlockSpec((1, GW), lambda i: (0, i))],
            out_specs=[pl.BlockSpec((GW, 2 * EMB_DIM), lambda i: (i, 0))],
            core_axis_name=("c", "s"),
            dimension_semantics=(pltpu.PARALLEL,),
        )(idx_hbm, out_hbm)

    return k(table, idx)


def _dense_body(num_ref, emb_ref, par_ref, dmat, rlo, bw0, bb0, bw1, bb1,
                bw2, bb2, w0bm, w0z, tb0, tw1, tb1, tw2, tb2, tw3, tb3,
                tw4, tb4, out_ref):
    f32 = jnp.float32
    x = num_ref[...]
    h = jnp.maximum(jnp.dot(x, bw0[...], preferred_element_type=f32) + bb0[...], 0.0)
    h = jnp.maximum(jnp.dot(h, bw1[...], preferred_element_type=f32) + bb1[...], 0.0)
    bm = jnp.maximum(jnp.dot(h, bw2[...], preferred_element_type=f32) + bb2[...], 0.0)

    keep = (jnp.dot(par_ref[...], dmat[...], preferred_element_type=f32)
            + rlo[...])
    masked = emb_ref[...] * keep
    bm_ext = jnp.concatenate([bm, jnp.zeros((BR, EMB_DIM), f32)], axis=1)
    flat = jnp.concatenate(
        [bm_ext, masked, jnp.zeros((BR, (NFP - NF) * 2 * EMB_DIM), f32)],
        axis=1)
    m3 = flat.reshape(BR, NFP, 2 * EMB_DIM)
    feats = m3[:, :, :EMB_DIM] + m3[:, :, EMB_DIM:]

    zcols = []
    for n in range(NF):
        zcols.append(jnp.sum(feats * feats[:, n:n + 1, :], axis=2))
    zflat = jnp.concatenate(zcols, axis=1)

    h = (jnp.dot(bm, w0bm[...], preferred_element_type=f32)
         + jnp.dot(zflat, w0z[...], preferred_element_type=f32) + tb0[...])
    h = jnp.maximum(h, 0.0)
    h = jnp.maximum(jnp.dot(h, tw1[...], preferred_element_type=f32) + tb1[...], 0.0)
    h = jnp.maximum(jnp.dot(h, tw2[...], preferred_element_type=f32) + tb2[...], 0.0)
    h = jnp.maximum(jnp.dot(h, tw3[...], preferred_element_type=f32) + tb3[...], 0.0)
    h = jnp.dot(h, tw4[...], preferred_element_type=f32) + tb4[...]
    out_ref[...] = jax.nn.sigmoid(h)


def _dense(numerical_input, emb, par, dmat, rlo, bw0, bb0, bw1, bb1, bw2,
           bb2, w0bm, w0z, tb0, tw1, tb1, tw2, tb2, tw3, tb3, tw4, tb4):
    n_blocks = B // BR

    def full(a):
        return pl.BlockSpec(a.shape, lambda i: tuple(0 for _ in a.shape))

    weights = (dmat, rlo, bw0, bb0, bw1, bb1, bw2, bb2, w0bm, w0z, tb0,
               tw1, tb1, tw2, tb2, tw3, tb3, tw4, tb4)
    return pl.pallas_call(
        _dense_body,
        grid=(n_blocks,),
        in_specs=[
            pl.BlockSpec((BR, NUM_DENSE), lambda i: (i, 0)),
            pl.BlockSpec((BR, N_CAT * 2 * EMB_DIM), lambda i: (i, 0)),
            pl.BlockSpec((BR, N_CAT), lambda i: (i, 0)),
        ] + [full(w) for w in weights],
        out_specs=pl.BlockSpec((BR, 1), lambda i: (i, 0)),
        out_shape=jax.ShapeDtypeStruct((B, 1), jnp.float32),
        compiler_params=pltpu.CompilerParams(
            dimension_semantics=("arbitrary",)),
    )(numerical_input, emb, par, *weights)


def kernel(numerical_input, categorical_inputs, emb_tables,
           bw0, bb0, bw1, bb1, bw2, bb2,
           tw0, tb0, tw1, tb1, tw2, tb2, tw3, tb3, tw4, tb4):
    cat = categorical_inputs.astype(jnp.int32)
    offs = (jnp.arange(N_CAT, dtype=jnp.int32) * VOCAB)[None, :]
    idx = cat + offs
    pair_idx = (idx >> 1).reshape(1, B * N_CAT)
    par = (idx & 1).astype(jnp.float32)
    table = emb_tables.reshape(N_CAT * VOCAB // 2, 2 * EMB_DIM)
    emb = _sc_gather(table, pair_idx).reshape(B, N_CAT * 2 * EMB_DIM)

    dmat_np = np.zeros((N_CAT, N_CAT * 2 * EMB_DIM), np.float32)
    rlo_np = np.zeros((1, N_CAT * 2 * EMB_DIM), np.float32)
    for n in range(N_CAT):
        dmat_np[n, n * 128:n * 128 + EMB_DIM] = -1.0
        dmat_np[n, n * 128 + EMB_DIM:(n + 1) * 128] = 1.0
        rlo_np[0, n * 128:n * 128 + EMB_DIM] = 1.0
    dmat = jnp.asarray(dmat_np)
    rlo = jnp.asarray(rlo_np)

    li, lj = np.tril_indices(NF, -1)
    w0bm = tw0[:EMB_DIM]
    w0z = jnp.zeros((NF * NFP, tw0.shape[1]), jnp.float32)
    w0z = w0z.at[li * NFP + lj].set(tw0[EMB_DIM:])

    def row(b):
        return b.reshape(1, -1)

    return _dense(numerical_input, emb, par, dmat, rlo, bw0, row(bb0),
                  bw1, row(bb1), bw2, row(bb2), w0bm, w0z, row(tb0),
                  tw1, row(tb1), tw2, row(tb2), tw3, row(tb3),
                  tw4, row(tb4))


# speedup vs baseline: 2.0047x; 2.0047x over previous
import functools

import jax
import jax.numpy as jnp
import numpy as np
from jax import lax
from jax.experimental import pallas as pl
from jax.experimental.pallas import tpu as pltpu
from jax.experimental.pallas import tpu_sc as plsc

B = 16384
NUM_DENSE = 13
N_CAT = 26
VOCAB = 100000
EMB_DIM = 64
NF = N_CAT + 1
NFP = 32
GW = 128
BR = 512


def _sc_gather(table, idx):
    n_idx = idx.shape[1]
    mesh = plsc.VectorSubcoreMesh(core_axis_name="c", subcore_axis_name="s")

    @functools.partial(
        pl.kernel,
        mesh=mesh,
        out_type=jax.ShapeDtypeStruct((n_idx, 2 * EMB_DIM), jnp.float32),
    )
    def k(table_hbm, idx_hbm, out_hbm):
        def body(i_vmem, o_vmem):
            pltpu.sync_copy(table_hbm.at[i_vmem.at[0]], o_vmem)

        pltpu.emit_pipeline(
            body,
            grid=(n_idx // GW,),
            in_specs=[pl.BlockSpec((1, GW), lambda i: (0, i))],
            out_specs=[pl.BlockSpec((GW, 2 * EMB_DIM), lambda i: (i, 0))],
            core_axis_name=("c", "s"),
            dimension_semantics=(pltpu.PARALLEL,),
        )(idx_hbm, out_hbm)

    return k(table, idx)


def _dense_body(num_ref, emb_ref, par_ref, dmat, rlo, bw0, bb0, bw1, bb1,
                bw2, bb2, w0bm, w0z, tb0, tw1, tb1, tw2, tb2, tw3, tb3,
                tw4, tb4, out_ref):
    f32, bf16 = jnp.float32, jnp.bfloat16

    def mm(a, w):
        return jax.lax.dot_general(
            a, w[...], (((1,), (0,)), ((), ())), preferred_element_type=f32)

    x = num_ref[...].astype(bf16)
    h = jnp.maximum(mm(x, bw0) + bb0[...], 0.0).astype(bf16)
    h = jnp.maximum(mm(h, bw1) + bb1[...], 0.0).astype(bf16)
    bm = jnp.maximum(mm(h, bw2) + bb2[...], 0.0)

    keep = jnp.dot(par_ref[...], dmat[...], preferred_element_type=f32) + rlo[...]
    masked = (emb_ref[...] * keep).astype(bf16)
    bm16 = bm.astype(bf16)
    bm_ext = jnp.concatenate([bm16, jnp.zeros((BR, EMB_DIM), bf16)], axis=1)
    flat = jnp.concatenate(
        [bm_ext, masked, jnp.zeros((BR, (NFP - NF) * 2 * EMB_DIM), bf16)],
        axis=1)
    m3 = flat.reshape(BR, NFP, 2 * EMB_DIM)
    feats = m3[:, :, :EMB_DIM] + m3[:, :, EMB_DIM:]

    z = jax.lax.dot_general(feats, feats, (((2,), (2,)), ((0,), (0,))),
                            preferred_element_type=f32)
    zflat = z.astype(bf16).reshape(BR, NFP * NFP)
    hz = jax.lax.dot_general(zflat, w0z[...], (((1,), (0,)), ((), ())),
                             preferred_element_type=f32)
    h = jnp.maximum(mm(bm16, w0bm) + hz + tb0[...], 0.0).astype(bf16)
    h = jnp.maximum(mm(h, tw1) + tb1[...], 0.0).astype(bf16)
    h = jnp.maximum(mm(h, tw2) + tb2[...], 0.0).astype(bf16)
    h = jnp.maximum(mm(h, tw3) + tb3[...], 0.0).astype(bf16)
    h = mm(h, tw4) + tb4[...]
    out_ref[...] = jax.nn.sigmoid(h)


def _dense(numerical_input, emb, par, dmat, rlo, bw0, bb0, bw1, bb1, bw2,
           bb2, w0bm, w0z, tb0, tw1, tb1, tw2, tb2, tw3, tb3, tw4, tb4):
    n_blocks = B // BR

    def full(a):
        return pl.BlockSpec(a.shape, lambda i: tuple(0 for _ in a.shape))

    weights = (dmat, rlo, bw0, bb0, bw1, bb1, bw2, bb2, w0bm, w0z, tb0,
               tw1, tb1, tw2, tb2, tw3, tb3, tw4, tb4)
    return pl.pallas_call(
        _dense_body,
        grid=(n_blocks,),
        in_specs=[
            pl.BlockSpec((BR, NUM_DENSE), lambda i: (i, 0)),
            pl.BlockSpec((BR, N_CAT * 2 * EMB_DIM), lambda i: (i, 0)),
            pl.BlockSpec((BR, N_CAT), lambda i: (i, 0)),
        ] + [full(w) for w in weights],
        out_specs=pl.BlockSpec((BR, 1), lambda i: (i, 0)),
        out_shape=jax.ShapeDtypeStruct((B, 1), jnp.float32),
        compiler_params=pltpu.CompilerParams(
            dimension_semantics=("arbitrary",)),
    )(numerical_input, emb, par, *weights)


def kernel(numerical_input, categorical_inputs, emb_tables,
           bw0, bb0, bw1, bb1, bw2, bb2,
           tw0, tb0, tw1, tb1, tw2, tb2, tw3, tb3, tw4, tb4):
    bf16 = jnp.bfloat16
    cat = categorical_inputs.astype(jnp.int32)
    offs = (jnp.arange(N_CAT, dtype=jnp.int32) * VOCAB)[None, :]
    idx = cat + offs
    pair_idx = (idx >> 1).reshape(1, B * N_CAT)
    par = (idx & 1).astype(bf16)
    table = emb_tables.reshape(N_CAT * VOCAB // 2, 2 * EMB_DIM)
    emb = _sc_gather(table, pair_idx).reshape(B, N_CAT * 2 * EMB_DIM)

    dmat_np = np.zeros((N_CAT, N_CAT * 2 * EMB_DIM), np.float32)
    rlo_np = np.zeros((1, N_CAT * 2 * EMB_DIM), np.float32)
    for n in range(N_CAT):
        dmat_np[n, n * 128:n * 128 + EMB_DIM] = -1.0
        dmat_np[n, n * 128 + EMB_DIM:(n + 1) * 128] = 1.0
        rlo_np[0, n * 128:n * 128 + EMB_DIM] = 1.0
    dmat = jnp.asarray(dmat_np, dtype=bf16)
    rlo = jnp.asarray(rlo_np)

    li, lj = np.tril_indices(NF, -1)
    w0bm = tw0[:EMB_DIM].astype(bf16)
    w0z = jnp.zeros((NFP * NFP, tw0.shape[1]), jnp.float32)
    w0z = w0z.at[li * NFP + lj].set(tw0[EMB_DIM:]).astype(bf16)

    def row(b):
        return b.reshape(1, -1)

    return _dense(numerical_input, emb, par, dmat, rlo,
                  bw0.astype(bf16), row(bb0), bw1.astype(bf16), row(bb1),
                  bw2.astype(bf16), row(bb2), w0bm, w0z, row(tb0),
                  tw1.astype(bf16), row(tb1), tw2.astype(bf16), row(tb2),
                  tw3.astype(bf16), row(tb3), tw4.astype(bf16), row(tb4))
